# baseline (speedup 1.0000x reference)
import functools
from typing import Sequence

import jax
import jax.numpy as jnp
from jax import lax
from jax.experimental import pallas as pl
from jax.experimental.pallas import tpu as pltpu
from jax.experimental.pallas import tpu_sc as plsc

_INTERPRET = False


def _conv_body(nparts, want_norms, has_recon, widths, E, O, B, eps, *refs):
    nin = nparts + 4 * nparts + 3 + (2 if has_recon else 0)
    ins = refs[:nin]
    outs = refs[nin:-4]
    h_buf, sum_s, mu_s, sd_s = refs[-4:]
    parts = ins[:nparts]
    slots = ins[nparts:nparts + 4 * nparts]
    Wt = ins[nparts + 4 * nparts]
    gamma = ins[nparts + 4 * nparts + 1]
    beta = ins[nparts + 4 * nparts + 2]
    ph = pl.program_id(0)
    i = pl.program_id(1)

    @pl.when(ph == 0)
    def _():
        def cut(r, q):
            return r[:, :widths[q]] if widths[q] < r.shape[1] else r[...]
        pieces = [jnp.concatenate([cut(p, q) for q, p in enumerate(parts)],
                                  axis=1) if nparts > 1 else cut(parts[0], 0)]
        ga = [jnp.concatenate([cut(slots[s * nparts + q], q)
                               for q in range(nparts)],
                              axis=1) if nparts > 1
              else cut(slots[s], 0)
              for s in range(4)]
        pieces += [jnp.minimum(ga[0], ga[1]), jnp.maximum(ga[0], ga[1]),
                   jnp.minimum(ga[2], ga[3]), jnp.maximum(ga[2], ga[3])]
        feat = jnp.concatenate(pieces, axis=1)
        h = jnp.dot(feat, Wt[...], preferred_element_type=jnp.float32)
        h_buf[pl.ds(i * B, B), :] = h
        s = jnp.sum(h, axis=0, keepdims=True)

        @pl.when(i == 0)
        def _():
            sum_s[...] = s

        @pl.when(i > 0)
        def _():
            sum_s[...] = sum_s[...] + s

    @pl.when(ph == 1)
    def _():
        @pl.when(i == 0)
        def _():
            mu_s[...] = sum_s[...] / E
        d = h_buf[pl.ds(i * B, B), :] - mu_s[...]
        s = jnp.sum(d * d, axis=0, keepdims=True)

        @pl.when(i == 0)
        def _():
            sum_s[...] = s

        @pl.when(i > 0)
        def _():
            sum_s[...] = sum_s[...] + s

    @pl.when(ph == 2)
    def _():
        @pl.when(i == 0)
        def _():
            sd_s[...] = jnp.sqrt(sum_s[...] / E + eps)
        h = h_buf[pl.ds(i * B, B), :]
        e = (h - mu_s[...]) / sd_s[...] * gamma[...] + beta[...]
        e = jnp.maximum(e, 0.0)
        if has_recon:
            Wrt, br = ins[nparts * 5 + 3], ins[nparts * 5 + 4]
            outs[0][...] = jnp.dot(e, Wrt[...],
                                   preferred_element_type=jnp.float32) + br[...]
        else:
            if O < 128:
                outs[0][...] = jnp.concatenate(
                    [e, jnp.zeros((B, 128 - O), jnp.float32)], axis=1)
            else:
                outs[0][...] = e
            if want_norms:
                outs[1][...] = jnp.sqrt(jnp.sum(e * e, axis=1, keepdims=True))


def _mesh_conv_tc(parts, slots, widths, Wt, gamma, beta, *, want_norms=False,
                  Wr_t=None, br=None, block=1024):
    E = parts[0].shape[0]
    O = Wt.shape[1]
    B = min(block, E)
    NB = E // B
    nparts = len(parts)
    flat_slots = [slots[s][q] for s in range(4) for q in range(nparts)]
    g2 = gamma.reshape(1, O)
    b2 = beta.reshape(1, O)

    def pspec(c):
        return pl.BlockSpec((B, c), lambda ph, i: (jnp.where(ph == 0, i, 0), 0))

    in_specs = [pspec(p.shape[1]) for p in parts]
    in_specs += [pspec(p.shape[1]) for p in flat_slots]
    in_specs += [pl.BlockSpec(Wt.shape, lambda ph, i: (0, 0)),
                 pl.BlockSpec((1, O), lambda ph, i: (0, 0)),
                 pl.BlockSpec((1, O), lambda ph, i: (0, 0))]
    operands = list(parts) + flat_slots + [Wt, g2, b2]
    if Wr_t is not None:
        Orr = Wr_t.shape[1]
        in_specs += [pl.BlockSpec(Wr_t.shape, lambda ph, i: (0, 0)),
                     pl.BlockSpec((1, Orr), lambda ph, i: (0, 0))]
        operands += [Wr_t, br.reshape(1, Orr)]
        out_shape = [jax.ShapeDtypeStruct((E, Orr), jnp.float32)]
        out_specs = [pl.BlockSpec((B, Orr), lambda ph, i: (jnp.where(ph == 2, i, 0), 0))]
    else:
        out_shape = [jax.ShapeDtypeStruct((E, 128), jnp.float32)]
        out_specs = [pl.BlockSpec((B, 128), lambda ph, i: (jnp.where(ph == 2, i, 0), 0))]
        if want_norms:
            out_shape.append(jax.ShapeDtypeStruct((E, 1), jnp.float32))
            out_specs.append(pl.BlockSpec((B, 1), lambda ph, i: (jnp.where(ph == 2, i, 0), 0)))

    res = pl.pallas_call(
        functools.partial(_conv_body, nparts, want_norms, Wr_t is not None,
                          tuple(widths), E, O, B, 1e-5),
        grid=(3, NB),
        in_specs=in_specs,
        out_specs=out_specs,
        out_shape=out_shape,
        scratch_shapes=[pltpu.VMEM((E, O), jnp.float32),
                        pltpu.VMEM((1, O), jnp.float32),
                        pltpu.VMEM((1, O), jnp.float32),
                        pltpu.VMEM((1, O), jnp.float32)],
        interpret=_INTERPRET,
    )(*operands)
    return res


def _rank_body(k, B, NB, nc_ref, nr_ref, out_ref, c_s):
    i = pl.program_id(0)
    j = pl.program_id(1)
    nc = nc_ref[...]
    nr = nr_ref[...]

    @pl.when(j < i)
    def _():
        c_s[...] = jnp.sum((nr >= nc).astype(jnp.int32), axis=1, keepdims=True)

    @pl.when(j > i)
    def _():
        c_s[...] = jnp.sum((nr > nc).astype(jnp.int32), axis=1, keepdims=True)

    @pl.when(j == i)
    def _():
        ii = lax.broadcasted_iota(jnp.int32, (B, 1), 0)
        jj = lax.broadcasted_iota(jnp.int32, (1, B), 1)
        cmp = (nr > nc) | ((nr == nc) & (jj < ii))
        c_s[...] = jnp.sum(cmp.astype(jnp.int32), axis=1, keepdims=True)

    @pl.when(j == 0)
    def _():
        out_ref[...] = c_s[...]

    @pl.when(j > 0)
    def _():
        out_ref[...] = out_ref[...] + c_s[...]

    @pl.when(j == NB - 1)
    def _():
        out_ref[...] = (out_ref[...] < k).astype(jnp.int32)


def _topk_mask_tc(norms_col, norms_row, k, B=1024):
    E = norms_col.shape[0]
    NB = E // B
    return pl.pallas_call(
        functools.partial(_rank_body, k, B, NB),
        grid=(NB, NB),
        in_specs=[pl.BlockSpec((B, 1), lambda i, j: (i, 0)),
                  pl.BlockSpec((1, B), lambda i, j: (0, j))],
        out_specs=pl.BlockSpec((B, 1), lambda i, j: (i, 0)),
        out_shape=jax.ShapeDtypeStruct((E, 1), jnp.int32),
        scratch_shapes=[pltpu.VMEM((B, 1), jnp.int32)],
        interpret=_INTERPRET,
    )(norms_col, norms_row)


_BIGI = 1 << 30


def _remap_body(B, E, mc_ref, mrb_ref, mrf_ref, remap_ref, src_ref):
    i = pl.program_id(0)
    mc = mc_ref[...] > 0
    mrb = mrb_ref[...] > 0
    mrf = mrf_ref[...] > 0
    ii = lax.broadcasted_iota(jnp.int32, (B, 1), 0)
    jj = lax.broadcasted_iota(jnp.int32, (1, B), 1)
    gi = i * B + ii
    gj = i * B + jj
    gjf = lax.broadcasted_iota(jnp.int32, (1, E), 1)
    before = mrf & (gjf < i * B)
    after = mrf & (gjf >= (i + 1) * B)
    pfx = jnp.sum(before.astype(jnp.int32))
    lprev = jnp.max(jnp.where(before, gjf, -1))
    rnext = jnp.min(jnp.where(after, gjf, _BIGI))
    le = mrb & (jj <= ii)
    cnt = pfx + jnp.sum(le.astype(jnp.int32), axis=1, keepdims=True)
    L = jnp.maximum(jnp.max(jnp.where(le, gj, -1), axis=1, keepdims=True), lprev)
    R = jnp.minimum(jnp.min(jnp.where(mrb & (jj >= ii), gj, _BIGI),
                            axis=1, keepdims=True), rnext)
    remap_ref[...] = jnp.where(mc, cnt - 1, -1)
    useL = (L >= 0) & ((R >= _BIGI) | ((gi - L) <= (R - gi)))
    src_ref[...] = jnp.where(useL, cnt - 1, cnt)


def _remap_src_tc(mask_col, mask_row, B=512):
    E = mask_col.shape[0]
    NB = E // B
    return pl.pallas_call(
        functools.partial(_remap_body, B, E),
        grid=(NB,),
        in_specs=[pl.BlockSpec((B, 1), lambda i: (i, 0)),
                  pl.BlockSpec((1, B), lambda i: (0, i)),
                  pl.BlockSpec((1, E), lambda i: (0, 0))],
        out_specs=[pl.BlockSpec((B, 1), lambda i: (i, 0)),
                   pl.BlockSpec((B, 1), lambda i: (i, 0))],
        out_shape=[jax.ShapeDtypeStruct((E, 1), jnp.int32),
                   jax.ShapeDtypeStruct((E, 1), jnp.int32)],
        interpret=_INTERPRET,
    )(mask_col, mask_row, mask_row)


def _keep_body(Bt, Bj, NBj, rm_ref, out_ref):
    t = pl.program_id(0)
    j = pl.program_id(1)
    rm = rm_ref[...]
    tidx = t * Bt + lax.broadcasted_iota(jnp.int32, (Bt, 1), 0)
    jidx = j * Bj + lax.broadcasted_iota(jnp.int32, (1, Bj), 1)
    c = jnp.sum(jnp.where(rm == tidx, jidx, 0), axis=1, keepdims=True)

    @pl.when(j == 0)
    def _():
        out_ref[...] = c

    @pl.when(j > 0)
    def _():
        out_ref[...] = out_ref[...] + c


def _keep_tc(remap_row, k, Bt=512, Bj=1024):
    E = remap_row.shape[1]
    NBt, NBj = k // Bt, E // Bj
    return pl.pallas_call(
        functools.partial(_keep_body, Bt, Bj, NBj),
        grid=(NBt, NBj),
        in_specs=[pl.BlockSpec((1, Bj), lambda t, j: (0, j))],
        out_specs=pl.BlockSpec((Bt, 1), lambda t, j: (t, 0)),
        out_shape=jax.ShapeDtypeStruct((k, 1), jnp.int32),
        interpret=_INTERPRET,
    )(remap_row)


def _heads_body(e_ref, wst, bs, wc1t, bc1, wc2t, bc2, sum_ref, log_ref):
    e = e_ref[...]
    sum_ref[...] = jnp.dot(e, wst[...], preferred_element_type=jnp.float32) + bs[...]
    g = jnp.max(e, axis=0, keepdims=True)
    h = jnp.maximum(jnp.dot(g, wc1t[...], preferred_element_type=jnp.float32)
                    + bc1[...], 0.0)
    log_ref[...] = jnp.dot(h, wc2t[...], preferred_element_type=jnp.float32) + bc2[...]


def _heads_tc(e2p, Ws, bs, Wc1, bc1, Wc2, bc2):
    k, C = e2p.shape
    F = Ws.shape[0]
    K = Wc2.shape[0]
    H = Wc1.shape[0]
    return pl.pallas_call(
        _heads_body,
        out_shape=[jax.ShapeDtypeStruct((k, F), jnp.float32),
                   jax.ShapeDtypeStruct((1, K), jnp.float32)],
        interpret=_INTERPRET,
    )(e2p, Ws.T, bs.reshape(1, F), Wc1.T, bc1.reshape(1, H), Wc2.T,
      bc2.reshape(1, K))


def _sc_mesh():
    return plsc.VectorSubcoreMesh(core_axis_name="c", subcore_axis_name="s",
                                  num_cores=2, num_subcores=16)


_NW = 32


def _sc_neighbor_gather(table, nbT, src=None):
    E = nbT.shape[1]
    Q = E // _NW
    NC = Q // 128
    D = table.shape[1]
    hs = src is not None
    ng = 5 if hs else 4

    out_type = [jax.ShapeDtypeStruct((E, D), jnp.float32)] * ng
    scratch = ([pltpu.VMEM((128,), jnp.int32)] * 4
               + [pltpu.VMEM((128,), jnp.int32)] * 4
               + [pltpu.VMEM((128, D), jnp.float32)] * ng
               + ([pltpu.VMEM((E,), jnp.int32)] if hs else [])
               + [pltpu.SemaphoreType.DMA] * 3)

    def body(*refs):
        p = 1 + 1 + (1 if hs else 0)
        t_ref, nbf_ref = refs[0], refs[1]
        src_ref = refs[2] if hs else None
        o_refs = refs[p:p + ng]; p += ng
        nbidx = refs[p:p + 4]; p += 4
        cidx = refs[p:p + 4]; p += 4
        rows = refs[p:p + ng]; p += ng
        if hs:
            sfull = refs[p]; p += 1
        semI, semG, semS = refs[p:p + 3]

        w = lax.axis_index("s") * 2 + lax.axis_index("c")
        if hs:
            pltpu.sync_copy(src_ref, sfull)
        for c in range(NC):
            base = w * Q + c * 128
            di = [pltpu.async_copy(nbf_ref.at[pl.ds(s * E + base, 128)],
                                   nbidx[s], semI) for s in range(4)]
            dg = []
            if hs:
                dg.append(pltpu.async_copy(
                    t_ref.at[sfull.at[pl.ds(base, 128)]], rows[4], semG))
            for s in range(4):
                di[s].wait()
                if hs:
                    for c16 in range(8):
                        i16 = nbidx[s][pl.ds(c16 * 16, 16)]
                        cidx[s][pl.ds(c16 * 16, 16)] = plsc.load_gather(
                            sfull, [i16])
                    idxr = cidx[s]
                else:
                    idxr = nbidx[s]
                dg.append(pltpu.async_copy(t_ref.at[idxr], rows[s], semG))
            for d in dg:
                d.wait()
            ds = [pltpu.async_copy(rows[s], o_refs[s].at[pl.ds(base, 128)],
                                   semS) for s in range(ng)]
            for d in ds:
                d.wait()

    fn = pl.kernel(body, out_type=out_type, mesh=_sc_mesh(),
                   scratch_types=scratch,
                   compiler_params=pltpu.CompilerParams(
                       needs_layout_passes=False))
    operands = [table, nbT.reshape(-1)] + ([src] if hs else [])
    flat = fn(*operands)
    if hs:
        return flat[4], list(flat[:4])
    return None, list(flat)


def _sc_narrow_gather(xflat, nbT, F):
    E = nbT.shape[1]
    Q = E // _NW
    NC = Q // 128
    out_type = [jax.ShapeDtypeStruct((E, 16), jnp.float32)] * 4
    scratch = ([pltpu.VMEM((F * E,), jnp.float32)]
               + [pltpu.VMEM((128,), jnp.int32)] * 4
               + [pltpu.VMEM((128, 16), jnp.float32)] * 4
               + [pltpu.SemaphoreType.DMA] * 2)

    def body(*refs):
        xf_ref, nbf_ref = refs[0], refs[1]
        o_refs = refs[2:6]
        xf = refs[6]
        nbidx = refs[7:11]
        obuf = refs[11:15]
        semI, semS = refs[15], refs[16]
        w = lax.axis_index("s") * 2 + lax.axis_index("c")
        pltpu.sync_copy(xf_ref, xf)
        iota = lax.iota(jnp.int32, 16)
        for c in range(NC):
            base = w * Q + c * 128
            di = [pltpu.async_copy(nbf_ref.at[pl.ds(s * E + base, 128)],
                                   nbidx[s], semI) for s in range(4)]
            for s in range(4):
                di[s].wait()
                for c16 in range(8):
                    i16 = nbidx[s][pl.ds(c16 * 16, 16)]
                    rows = c16 * 16 + iota
                    for col in range(F):
                        v = plsc.load_gather(xf, [i16 * F + col])
                        plsc.store_scatter(obuf[s], [rows, jnp.full(
                            (16,), col, jnp.int32)], v)
            ds = [pltpu.async_copy(obuf[s], o_refs[s].at[pl.ds(base, 128)],
                                   semS) for s in range(4)]
            for d in ds:
                d.wait()

    fn = pl.kernel(body, out_type=out_type, mesh=_sc_mesh(),
                   scratch_types=scratch,
                   compiler_params=pltpu.CompilerParams(
                       needs_layout_passes=False))
    return list(fn(xflat, nbT.reshape(-1)))


def _sc_pool_gather(h, nbf, keep, remap, k, want_nb):
    E = h.shape[0]
    Qk = k // _NW
    cQ = min(Qk, 128)
    NC = Qk // cQ

    out_type = [jax.ShapeDtypeStruct((k, 128), jnp.float32)]
    if want_nb:
        out_type.append(jax.ShapeDtypeStruct((4 * k,), jnp.int32))
    scratch = [pltpu.VMEM((cQ,), jnp.int32),
               pltpu.VMEM((cQ, 128), jnp.float32),
               pltpu.SemaphoreType.DMA, pltpu.SemaphoreType.DMA]
    if want_nb:
        scratch = ([pltpu.VMEM((E,), jnp.int32),
                    pltpu.VMEM((4 * E,), jnp.int32)]
                   + [pltpu.VMEM((cQ,), jnp.int32)] * 4 + scratch)

    def body(*refs):
        if want_nb:
            (h_ref, nbf_ref, keep_ref, remap_ref, hp_ref, nbo_ref,
             rfull, nbsf, o0, o1, o2, o3, kv, hrows, semG, semS) = refs
            nbo = [o0, o1, o2, o3]
        else:
            h_ref, keep_ref, hp_ref, kv, hrows, semG, semS = refs
        w = lax.axis_index("s") * 2 + lax.axis_index("c")
        if want_nb:
            pltpu.sync_copy(remap_ref, rfull)
            pltpu.sync_copy(nbf_ref, nbsf)
        iota = lax.iota(jnp.int32, 16)
        for c in range(NC):
            base = w * Qk + c * cQ
            pltpu.sync_copy(keep_ref.at[pl.ds(base, cQ)], kv)
            dg = pltpu.async_copy(h_ref.at[kv], hrows, semG)
            if want_nb:
                for c16 in range(cQ // 16):
                    k16 = kv[pl.ds(c16 * 16, 16)]
                    t16 = base + c16 * 16 + iota
                    for s in range(4):
                        nv = plsc.load_gather(nbsf, [k16 + s * E])
                        rv = plsc.load_gather(rfull, [nv])
                        nbo[s][pl.ds(c16 * 16, 16)] = jnp.where(rv < 0, t16, rv)
            dg.wait()
            dst = [pltpu.async_copy(hrows, hp_ref.at[pl.ds(base, cQ)], semS)]
            if want_nb:
                dst += [pltpu.async_copy(nbo[s],
                                         nbo_ref.at[pl.ds(s * k + base, cQ)],
                                         semS) for s in range(4)]
            for d in dst:
                d.wait()

    fn = pl.kernel(body, out_type=out_type, mesh=_sc_mesh(),
                   scratch_types=scratch,
                   compiler_params=pltpu.CompilerParams(
                       needs_layout_passes=False))
    if want_nb:
        return fn(h, nbf, keep, remap)
    res = fn(h, keep)
    return (res[0] if isinstance(res, (list, tuple)) else res), None


def _expand_wt(W, widths, padded, opad=128):
    O = W.shape[0]
    CT = sum(widths)
    blocks = []
    for s in range(5):
        off = s * CT
        for w, p in zip(widths, padded):
            blk = W[:, off:off + w].T
            if p > w:
                blk = jnp.concatenate([blk, jnp.zeros((p - w, O), W.dtype)], axis=0)
            blocks.append(blk)
            off += w
    Wt = jnp.concatenate(blocks, axis=0)
    if opad > O:
        Wt = jnp.concatenate([Wt, jnp.zeros((Wt.shape[0], opad - O), W.dtype)],
                             axis=1)
    return Wt


def _pad1(v, n=128):
    return jnp.concatenate([v, jnp.zeros((n - v.shape[0],), v.dtype)])


def kernel(x, nb, W1, g1, b1, W2, g2, b2, Ws, bs, Wd2, gd2, bd2,
           Wd1, gd1, bd1, Wr, br, Wc1, bc1, Wc2, bc2):
    E, F = x.shape
    t1, t2 = max(E // 2, 1), max(E // 8, 1)
    nb = nb.astype(jnp.int32)
    nbc = jnp.clip(nb, 0, E - 1)
    nbT = nbc.T
    nbc_flat = nbc.reshape(-1)
    xpad16 = jnp.concatenate([x, jnp.zeros((E, 16 - F), x.dtype)], axis=1)

    W1t, W2t, Wd2t, Wd1t, Wrt = W1.T, W2.T, Wd2.T, Wd1.T, Wr.T

    gx_slots = _sc_narrow_gather(x.reshape(-1), nbT, F)
    e1, n1 = _mesh_conv_tc([xpad16], [[s] for s in gx_slots], [F], W1t,
                           g1, b1, want_norms=True)

    m1 = _topk_mask_tc(n1, n1.reshape(1, E), t1)
    r1, s1 = _remap_src_tc(m1, m1.reshape(1, E))
    remap1 = r1.reshape(E)
    src1 = s1.reshape(E)
    keep1 = _keep_tc(r1.reshape(1, E), t1).reshape(t1)
    hp1, nb1f = _sc_pool_gather(e1, nbT.reshape(-1), keep1, remap1, t1, True)
    e1p = hp1
    nb1T = nb1f.reshape(4, t1)

    _, ge_slots = _sc_neighbor_gather(hp1, nb1T)
    e2, n2 = _mesh_conv_tc([e1p], [[s] for s in ge_slots], [64], W2t, g2, b2,
                           want_norms=True)

    m2 = _topk_mask_tc(n2, n2.reshape(1, t1), t2)
    r2, s2 = _remap_src_tc(m2, m2.reshape(1, t1))
    remap2 = r2.reshape(t1)
    src2 = s2.reshape(t1)
    keep2 = _keep_tc(r2.reshape(1, t1), t2).reshape(t2)
    hp2, _ = _sc_pool_gather(e2, None, keep2, remap2, t2, False)
    e2p = hp2

    summary, logits = _heads_tc(e2p, Ws, bs, Wc1, bc1, Wc2, bc2)

    u2, gu2_slots = _sc_neighbor_gather(hp2, nb1T, src=src2)
    d2 = _mesh_conv_tc([u2, e1p],
                       [[a, b] for a, b in zip(gu2_slots, ge_slots)],
                       [128, 64], Wd2t, gd2, bd2)[0]

    u1, gu1_slots = _sc_neighbor_gather(d2, nbT, src=src1)
    recon = _mesh_conv_tc([u1, xpad16],
                          [[a, b] for a, b in zip(gu1_slots, gx_slots)],
                          [64, F], Wd1t, gd1, bd1, Wr_t=Wrt, br=br)[0]

    return summary, recon, logits


# speedup vs baseline: 1.0193x; 1.0193x over previous
import functools
from typing import Sequence

import jax
import jax.numpy as jnp
from jax import lax
from jax.experimental import pallas as pl
from jax.experimental.pallas import tpu as pltpu
from jax.experimental.pallas import tpu_sc as plsc

_INTERPRET = False


def _conv_body(nparts, want_norms, has_recon, widths, E, O, B, eps, *refs):
    nin = nparts + 4 * nparts + 3 + (2 if has_recon else 0)
    ins = refs[:nin]
    outs = refs[nin:-4]
    h_buf, sum_s, mu_s, sd_s = refs[-4:]
    parts = ins[:nparts]
    slots = ins[nparts:nparts + 4 * nparts]
    Wt = ins[nparts + 4 * nparts]
    gamma = ins[nparts + 4 * nparts + 1]
    beta = ins[nparts + 4 * nparts + 2]
    ph = pl.program_id(0)
    i = pl.program_id(1)

    @pl.when(ph == 0)
    def _():
        def cut(r, q):
            return r[:, :widths[q]] if widths[q] < r.shape[1] else r[...]
        pieces = [jnp.concatenate([cut(p, q) for q, p in enumerate(parts)],
                                  axis=1) if nparts > 1 else cut(parts[0], 0)]
        ga = [jnp.concatenate([cut(slots[s * nparts + q], q)
                               for q in range(nparts)],
                              axis=1) if nparts > 1
              else cut(slots[s], 0)
              for s in range(4)]
        pieces += [jnp.minimum(ga[0], ga[1]), jnp.maximum(ga[0], ga[1]),
                   jnp.minimum(ga[2], ga[3]), jnp.maximum(ga[2], ga[3])]
        feat = jnp.concatenate(pieces, axis=1)
        h = jnp.dot(feat, Wt[...], preferred_element_type=jnp.float32)
        h_buf[pl.ds(i * B, B), :] = h
        s = jnp.sum(h, axis=0, keepdims=True)

        @pl.when(i == 0)
        def _():
            sum_s[...] = s

        @pl.when(i > 0)
        def _():
            sum_s[...] = sum_s[...] + s

    @pl.when(ph == 1)
    def _():
        @pl.when(i == 0)
        def _():
            mu_s[...] = sum_s[...] / E
        d = h_buf[pl.ds(i * B, B), :] - mu_s[...]
        s = jnp.sum(d * d, axis=0, keepdims=True)

        @pl.when(i == 0)
        def _():
            sum_s[...] = s

        @pl.when(i > 0)
        def _():
            sum_s[...] = sum_s[...] + s

    @pl.when(ph == 2)
    def _():
        @pl.when(i == 0)
        def _():
            sd_s[...] = jnp.sqrt(sum_s[...] / E + eps)
        h = h_buf[pl.ds(i * B, B), :]
        e = (h - mu_s[...]) / sd_s[...] * gamma[...] + beta[...]
        e = jnp.maximum(e, 0.0)
        if has_recon:
            Wrt, br = ins[nparts * 5 + 3], ins[nparts * 5 + 4]
            outs[0][...] = jnp.dot(e, Wrt[...],
                                   preferred_element_type=jnp.float32) + br[...]
        else:
            if O < 128:
                outs[0][...] = jnp.concatenate(
                    [e, jnp.zeros((B, 128 - O), jnp.float32)], axis=1)
            else:
                outs[0][...] = e
            if want_norms:
                outs[1][...] = jnp.sqrt(jnp.sum(e * e, axis=1, keepdims=True))


def _mesh_conv_tc(parts, slots, widths, Wt, gamma, beta, *, want_norms=False,
                  Wr_t=None, br=None, block=1024):
    E = parts[0].shape[0]
    O = Wt.shape[1]
    B = min(block, E)
    NB = E // B
    nparts = len(parts)
    flat_slots = [slots[s][q] for s in range(4) for q in range(nparts)]
    g2 = gamma.reshape(1, O)
    b2 = beta.reshape(1, O)

    def pspec(c):
        return pl.BlockSpec((B, c), lambda ph, i: (jnp.where(ph == 0, i, 0), 0))

    in_specs = [pspec(p.shape[1]) for p in parts]
    in_specs += [pspec(p.shape[1]) for p in flat_slots]
    in_specs += [pl.BlockSpec(Wt.shape, lambda ph, i: (0, 0)),
                 pl.BlockSpec((1, O), lambda ph, i: (0, 0)),
                 pl.BlockSpec((1, O), lambda ph, i: (0, 0))]
    operands = list(parts) + flat_slots + [Wt, g2, b2]
    if Wr_t is not None:
        Orr = Wr_t.shape[1]
        in_specs += [pl.BlockSpec(Wr_t.shape, lambda ph, i: (0, 0)),
                     pl.BlockSpec((1, Orr), lambda ph, i: (0, 0))]
        operands += [Wr_t, br.reshape(1, Orr)]
        out_shape = [jax.ShapeDtypeStruct((E, Orr), jnp.float32)]
        out_specs = [pl.BlockSpec((B, Orr), lambda ph, i: (jnp.where(ph == 2, i, 0), 0))]
    else:
        out_shape = [jax.ShapeDtypeStruct((E, 128), jnp.float32)]
        out_specs = [pl.BlockSpec((B, 128), lambda ph, i: (jnp.where(ph == 2, i, 0), 0))]
        if want_norms:
            out_shape.append(jax.ShapeDtypeStruct((E, 1), jnp.float32))
            out_specs.append(pl.BlockSpec((B, 1), lambda ph, i: (jnp.where(ph == 2, i, 0), 0)))

    res = pl.pallas_call(
        functools.partial(_conv_body, nparts, want_norms, Wr_t is not None,
                          tuple(widths), E, O, B, 1e-5),
        grid=(3, NB),
        in_specs=in_specs,
        out_specs=out_specs,
        out_shape=out_shape,
        scratch_shapes=[pltpu.VMEM((E, O), jnp.float32),
                        pltpu.VMEM((1, O), jnp.float32),
                        pltpu.VMEM((1, O), jnp.float32),
                        pltpu.VMEM((1, O), jnp.float32)],
        interpret=_INTERPRET,
    )(*operands)
    return res


def _rank_body(k, B, NB, nc_ref, nr_ref, out_ref, c_s):
    i = pl.program_id(0)
    j = pl.program_id(1)
    nc = nc_ref[...]
    nr = nr_ref[...]

    @pl.when(j < i)
    def _():
        c_s[...] = jnp.sum((nr >= nc).astype(jnp.int32), axis=1, keepdims=True)

    @pl.when(j > i)
    def _():
        c_s[...] = jnp.sum((nr > nc).astype(jnp.int32), axis=1, keepdims=True)

    @pl.when(j == i)
    def _():
        ii = lax.broadcasted_iota(jnp.int32, (B, 1), 0)
        jj = lax.broadcasted_iota(jnp.int32, (1, B), 1)
        cmp = (nr > nc) | ((nr == nc) & (jj < ii))
        c_s[...] = jnp.sum(cmp.astype(jnp.int32), axis=1, keepdims=True)

    @pl.when(j == 0)
    def _():
        out_ref[...] = c_s[...]

    @pl.when(j > 0)
    def _():
        out_ref[...] = out_ref[...] + c_s[...]

    @pl.when(j == NB - 1)
    def _():
        out_ref[...] = (out_ref[...] < k).astype(jnp.int32)


def _topk_mask_tc(norms_col, norms_row, k, B=1024):
    E = norms_col.shape[0]
    NB = E // B
    return pl.pallas_call(
        functools.partial(_rank_body, k, B, NB),
        grid=(NB, NB),
        in_specs=[pl.BlockSpec((B, 1), lambda i, j: (i, 0)),
                  pl.BlockSpec((1, B), lambda i, j: (0, j))],
        out_specs=pl.BlockSpec((B, 1), lambda i, j: (i, 0)),
        out_shape=jax.ShapeDtypeStruct((E, 1), jnp.int32),
        scratch_shapes=[pltpu.VMEM((B, 1), jnp.int32)],
        interpret=_INTERPRET,
    )(norms_col, norms_row)


_BIGI = 1 << 30


def _remap_body(B, E, mc_ref, mrb_ref, mrf_ref, remap_ref, src_ref):
    i = pl.program_id(0)
    mc = mc_ref[...] > 0
    mrb = mrb_ref[...] > 0
    mrf = mrf_ref[...] > 0
    ii = lax.broadcasted_iota(jnp.int32, (B, 1), 0)
    jj = lax.broadcasted_iota(jnp.int32, (1, B), 1)
    gi = i * B + ii
    gj = i * B + jj
    gjf = lax.broadcasted_iota(jnp.int32, (1, E), 1)
    before = mrf & (gjf < i * B)
    after = mrf & (gjf >= (i + 1) * B)
    pfx = jnp.sum(before.astype(jnp.int32))
    lprev = jnp.max(jnp.where(before, gjf, -1))
    rnext = jnp.min(jnp.where(after, gjf, _BIGI))
    le = mrb & (jj <= ii)
    cnt = pfx + jnp.sum(le.astype(jnp.int32), axis=1, keepdims=True)
    L = jnp.maximum(jnp.max(jnp.where(le, gj, -1), axis=1, keepdims=True), lprev)
    R = jnp.minimum(jnp.min(jnp.where(mrb & (jj >= ii), gj, _BIGI),
                            axis=1, keepdims=True), rnext)
    remap_ref[...] = jnp.where(mc, cnt - 1, -1)
    useL = (L >= 0) & ((R >= _BIGI) | ((gi - L) <= (R - gi)))
    src_ref[...] = jnp.where(useL, cnt - 1, cnt)


def _remap_src_tc(mask_col, mask_row, B=512):
    E = mask_col.shape[0]
    NB = E // B
    return pl.pallas_call(
        functools.partial(_remap_body, B, E),
        grid=(NB,),
        in_specs=[pl.BlockSpec((B, 1), lambda i: (i, 0)),
                  pl.BlockSpec((1, B), lambda i: (0, i)),
                  pl.BlockSpec((1, E), lambda i: (0, 0))],
        out_specs=[pl.BlockSpec((B, 1), lambda i: (i, 0)),
                   pl.BlockSpec((B, 1), lambda i: (i, 0))],
        out_shape=[jax.ShapeDtypeStruct((E, 1), jnp.int32),
                   jax.ShapeDtypeStruct((E, 1), jnp.int32)],
        interpret=_INTERPRET,
    )(mask_col, mask_row, mask_row)


def _keep_body(Bt, Bj, NBj, rm_ref, out_ref):
    t = pl.program_id(0)
    j = pl.program_id(1)
    rm = rm_ref[...]
    tidx = t * Bt + lax.broadcasted_iota(jnp.int32, (Bt, 1), 0)
    jidx = j * Bj + lax.broadcasted_iota(jnp.int32, (1, Bj), 1)
    c = jnp.sum(jnp.where(rm == tidx, jidx, 0), axis=1, keepdims=True)

    @pl.when(j == 0)
    def _():
        out_ref[...] = c

    @pl.when(j > 0)
    def _():
        out_ref[...] = out_ref[...] + c


def _keep_tc(remap_row, k, Bt=512, Bj=1024):
    E = remap_row.shape[1]
    NBt, NBj = k // Bt, E // Bj
    return pl.pallas_call(
        functools.partial(_keep_body, Bt, Bj, NBj),
        grid=(NBt, NBj),
        in_specs=[pl.BlockSpec((1, Bj), lambda t, j: (0, j))],
        out_specs=pl.BlockSpec((Bt, 1), lambda t, j: (t, 0)),
        out_shape=jax.ShapeDtypeStruct((k, 1), jnp.int32),
        interpret=_INTERPRET,
    )(remap_row)


def _heads_body(e_ref, wst, bs, wc1t, bc1, wc2t, bc2, sum_ref, log_ref):
    e = e_ref[...]
    sum_ref[...] = jnp.dot(e, wst[...], preferred_element_type=jnp.float32) + bs[...]
    g = jnp.max(e, axis=0, keepdims=True)
    h = jnp.maximum(jnp.dot(g, wc1t[...], preferred_element_type=jnp.float32)
                    + bc1[...], 0.0)
    log_ref[...] = jnp.dot(h, wc2t[...], preferred_element_type=jnp.float32) + bc2[...]


def _heads_tc(e2p, Ws, bs, Wc1, bc1, Wc2, bc2):
    k, C = e2p.shape
    F = Ws.shape[0]
    K = Wc2.shape[0]
    H = Wc1.shape[0]
    return pl.pallas_call(
        _heads_body,
        out_shape=[jax.ShapeDtypeStruct((k, F), jnp.float32),
                   jax.ShapeDtypeStruct((1, K), jnp.float32)],
        interpret=_INTERPRET,
    )(e2p, Ws.T, bs.reshape(1, F), Wc1.T, bc1.reshape(1, H), Wc2.T,
      bc2.reshape(1, K))


def _sc_mesh():
    return plsc.VectorSubcoreMesh(core_axis_name="c", subcore_axis_name="s",
                                  num_cores=2, num_subcores=16)


_NW = 32


def _sc_neighbor_gather(table, nbT, src=None):
    E = nbT.shape[1]
    Q = E // _NW
    NC = Q // 128
    D = table.shape[1]
    hs = src is not None
    ng = 5 if hs else 4

    out_type = [jax.ShapeDtypeStruct((E, D), jnp.float32)] * ng
    scratch = ([pltpu.VMEM((128,), jnp.int32)] * 4
               + [pltpu.VMEM((128,), jnp.int32)] * 4
               + [pltpu.VMEM((128, D), jnp.float32)] * ng
               + ([pltpu.VMEM((E,), jnp.int32)] if hs else [])
               + [pltpu.SemaphoreType.DMA] * 3)

    def body(*refs):
        p = 1 + 1 + (1 if hs else 0)
        t_ref, nbf_ref = refs[0], refs[1]
        src_ref = refs[2] if hs else None
        o_refs = refs[p:p + ng]; p += ng
        nbidx = refs[p:p + 4]; p += 4
        cidx = refs[p:p + 4]; p += 4
        rows = refs[p:p + ng]; p += ng
        if hs:
            sfull = refs[p]; p += 1
        semI, semG, semS = refs[p:p + 3]

        w = lax.axis_index("s") * 2 + lax.axis_index("c")
        if hs:
            pltpu.sync_copy(src_ref, sfull)
        for c in range(NC):
            base = w * Q + c * 128
            di = [pltpu.async_copy(nbf_ref.at[pl.ds(s * E + base, 128)],
                                   nbidx[s], semI) for s in range(4)]
            dg = []
            if hs:
                dg.append(pltpu.async_copy(
                    t_ref.at[sfull.at[pl.ds(base, 128)]], rows[4], semG))
            for s in range(4):
                di[s].wait()
                if hs:
                    for c16 in range(8):
                        i16 = nbidx[s][pl.ds(c16 * 16, 16)]
                        cidx[s][pl.ds(c16 * 16, 16)] = plsc.load_gather(
                            sfull, [i16])
                    idxr = cidx[s]
                else:
                    idxr = nbidx[s]
                dg.append(pltpu.async_copy(t_ref.at[idxr], rows[s], semG))
            for d in dg:
                d.wait()
            ds = [pltpu.async_copy(rows[s], o_refs[s].at[pl.ds(base, 128)],
                                   semS) for s in range(ng)]
            for d in ds:
                d.wait()

    fn = pl.kernel(body, out_type=out_type, mesh=_sc_mesh(),
                   scratch_types=scratch,
                   compiler_params=pltpu.CompilerParams(
                       needs_layout_passes=False))
    operands = [table, nbT.reshape(-1)] + ([src] if hs else [])
    flat = fn(*operands)
    if hs:
        return flat[4], list(flat[:4])
    return None, list(flat)


def _sc_pool_gather(h, nbf, keep, remap, k, want_nb):
    E = h.shape[0]
    Qk = k // _NW
    cQ = min(Qk, 128)
    NC = Qk // cQ

    out_type = [jax.ShapeDtypeStruct((k, 128), jnp.float32)]
    if want_nb:
        out_type.append(jax.ShapeDtypeStruct((4 * k,), jnp.int32))
    scratch = [pltpu.VMEM((cQ,), jnp.int32),
               pltpu.VMEM((cQ, 128), jnp.float32),
               pltpu.SemaphoreType.DMA, pltpu.SemaphoreType.DMA]
    if want_nb:
        scratch = ([pltpu.VMEM((E,), jnp.int32),
                    pltpu.VMEM((4 * E,), jnp.int32)]
                   + [pltpu.VMEM((cQ,), jnp.int32)] * 4 + scratch)

    def body(*refs):
        if want_nb:
            (h_ref, nbf_ref, keep_ref, remap_ref, hp_ref, nbo_ref,
             rfull, nbsf, o0, o1, o2, o3, kv, hrows, semG, semS) = refs
            nbo = [o0, o1, o2, o3]
        else:
            h_ref, keep_ref, hp_ref, kv, hrows, semG, semS = refs
        w = lax.axis_index("s") * 2 + lax.axis_index("c")
        if want_nb:
            pltpu.sync_copy(remap_ref, rfull)
            pltpu.sync_copy(nbf_ref, nbsf)
        iota = lax.iota(jnp.int32, 16)
        for c in range(NC):
            base = w * Qk + c * cQ
            pltpu.sync_copy(keep_ref.at[pl.ds(base, cQ)], kv)
            dg = pltpu.async_copy(h_ref.at[kv], hrows, semG)
            if want_nb:
                for c16 in range(cQ // 16):
                    k16 = kv[pl.ds(c16 * 16, 16)]
                    t16 = base + c16 * 16 + iota
                    for s in range(4):
                        nv = plsc.load_gather(nbsf, [k16 + s * E])
                        rv = plsc.load_gather(rfull, [nv])
                        nbo[s][pl.ds(c16 * 16, 16)] = jnp.where(rv < 0, t16, rv)
            dg.wait()
            dst = [pltpu.async_copy(hrows, hp_ref.at[pl.ds(base, cQ)], semS)]
            if want_nb:
                dst += [pltpu.async_copy(nbo[s],
                                         nbo_ref.at[pl.ds(s * k + base, cQ)],
                                         semS) for s in range(4)]
            for d in dst:
                d.wait()

    fn = pl.kernel(body, out_type=out_type, mesh=_sc_mesh(),
                   scratch_types=scratch,
                   compiler_params=pltpu.CompilerParams(
                       needs_layout_passes=False))
    if want_nb:
        return fn(h, nbf, keep, remap)
    res = fn(h, keep)
    return (res[0] if isinstance(res, (list, tuple)) else res), None


def _expand_wt(W, widths, padded, opad=128):
    O = W.shape[0]
    CT = sum(widths)
    blocks = []
    for s in range(5):
        off = s * CT
        for w, p in zip(widths, padded):
            blk = W[:, off:off + w].T
            if p > w:
                blk = jnp.concatenate([blk, jnp.zeros((p - w, O), W.dtype)], axis=0)
            blocks.append(blk)
            off += w
    Wt = jnp.concatenate(blocks, axis=0)
    if opad > O:
        Wt = jnp.concatenate([Wt, jnp.zeros((Wt.shape[0], opad - O), W.dtype)],
                             axis=1)
    return Wt


def _pad1(v, n=128):
    return jnp.concatenate([v, jnp.zeros((n - v.shape[0],), v.dtype)])


def kernel(x, nb, W1, g1, b1, W2, g2, b2, Ws, bs, Wd2, gd2, bd2,
           Wd1, gd1, bd1, Wr, br, Wc1, bc1, Wc2, bc2):
    E, F = x.shape
    t1, t2 = max(E // 2, 1), max(E // 8, 1)
    nb = nb.astype(jnp.int32)
    nbc = jnp.clip(nb, 0, E - 1)
    nbT = nbc.T
    nbc_flat = nbc.reshape(-1)
    xpad = jnp.concatenate([x, jnp.zeros((E, 128 - F), x.dtype)], axis=1)

    W1t, W2t, Wd2t, Wd1t, Wrt = W1.T, W2.T, Wd2.T, Wd1.T, Wr.T

    _, gx_slots = _sc_neighbor_gather(xpad, nbT)
    e1, n1 = _mesh_conv_tc([xpad], [[s] for s in gx_slots], [F], W1t,
                           g1, b1, want_norms=True)

    m1 = _topk_mask_tc(n1, n1.reshape(1, E), t1)
    r1, s1 = _remap_src_tc(m1, m1.reshape(1, E))
    remap1 = r1.reshape(E)
    src1 = s1.reshape(E)
    keep1 = _keep_tc(r1.reshape(1, E), t1).reshape(t1)
    hp1, nb1f = _sc_pool_gather(e1, nbT.reshape(-1), keep1, remap1, t1, True)
    e1p = hp1
    nb1T = nb1f.reshape(4, t1)

    _, ge_slots = _sc_neighbor_gather(hp1, nb1T)
    e2, n2 = _mesh_conv_tc([e1p], [[s] for s in ge_slots], [64], W2t, g2, b2,
                           want_norms=True)

    m2 = _topk_mask_tc(n2, n2.reshape(1, t1), t2)
    r2, s2 = _remap_src_tc(m2, m2.reshape(1, t1))
    remap2 = r2.reshape(t1)
    src2 = s2.reshape(t1)
    keep2 = _keep_tc(r2.reshape(1, t1), t2).reshape(t2)
    hp2, _ = _sc_pool_gather(e2, None, keep2, remap2, t2, False)
    e2p = hp2

    summary, logits = _heads_tc(e2p, Ws, bs, Wc1, bc1, Wc2, bc2)

    u2, gu2_slots = _sc_neighbor_gather(hp2, nb1T, src=src2)
    d2 = _mesh_conv_tc([u2, e1p],
                       [[a, b] for a, b in zip(gu2_slots, ge_slots)],
                       [128, 64], Wd2t, gd2, bd2)[0]

    u1, gu1_slots = _sc_neighbor_gather(d2, nbT, src=src1)
    recon = _mesh_conv_tc([u1, xpad],
                          [[a, b] for a, b in zip(gu1_slots, gx_slots)],
                          [64, F], Wd1t, gd1, bd1, Wr_t=Wrt, br=br)[0]

    return summary, recon, logits
